# baseline (speedup 1.0000x reference)
import functools

import jax
import jax.numpy as jnp
from jax import lax
from jax.experimental import pallas as pl
from jax.experimental.pallas import tpu as pltpu
from jax.experimental.pallas import tpu_sc as plsc

V = 8192
D = 8192
R = 2048
NC = 2
NS = 16
NW = NC * NS
K_SC = 1024
N_TC = R - K_SC
RPW = K_SC // NW
CH = 2
NCH = RPW // CH
L = 16
NBUF = 2


def _sc_body(x_hbm, sim_hbm, idx_hbm, out_hbm,
             idx_v, simbuf, xbuf, res,
             sem_s0, sem_s1, sem_x0, sem_x1):
    sem_s = (sem_s0, sem_s1)
    sem_x = (sem_x0, sem_x1)
    wid = lax.axis_index("s") * NC + lax.axis_index("c")
    base = wid * RPW

    pltpu.sync_copy(idx_hbm.at[wid], idx_v)

    def issue(c, buf):
        pltpu.async_copy(sim_hbm.at[idx_v.at[c]], simbuf.at[buf], sem_s[buf])
        pltpu.async_copy(x_hbm.at[pl.ds(base + c * CH, CH)], xbuf.at[buf],
                         sem_x[buf])

    def wait_chunk(c, buf):
        pltpu.make_async_copy(sim_hbm.at[idx_v.at[c]], simbuf.at[buf],
                              sem_s[buf]).wait()
        pltpu.make_async_copy(x_hbm.at[pl.ds(base + c * CH, CH)],
                              xbuf.at[buf], sem_x[buf]).wait()

    def compute(c, buf):
        for r in range(CH):
            row = c * CH + r

            def body(i, carry):
                sxr, sxx, srr = carry
                off = i * L
                xv = xbuf[buf, r, pl.ds(off, L)]
                rv = simbuf[buf, r, pl.ds(off, L)]
                return (sxr + xv * rv, sxx + xv * xv, srr + rv * rv)

            z = jnp.zeros((L,), jnp.float32)
            sxr, sxx, srr = lax.fori_loop(0, D // L, body, (z, z, z),
                                          unroll=8)
            res[0, row] = sxr
            res[1, row] = sxx
            res[2, row] = srr

    issue(0, 0)
    issue(1, 1)
    for c in range(NCH):
        buf = c % NBUF
        wait_chunk(c, buf)
        compute(c, buf)
        if c + NBUF < NCH:
            issue(c + NBUF, buf)

    pltpu.sync_copy(res, out_hbm.at[wid])


@functools.partial(
    pl.kernel,
    out_type=jax.ShapeDtypeStruct((NW, 3, RPW, L), jnp.float32),
    mesh=plsc.VectorSubcoreMesh(core_axis_name="c", subcore_axis_name="s"),
    scratch_types=[
        pltpu.VMEM((NCH, CH), jnp.int32),
        pltpu.VMEM((NBUF, CH, D), jnp.float32),
        pltpu.VMEM((NBUF, CH, D), jnp.float32),
        pltpu.VMEM((3, RPW, L), jnp.float32),
        pltpu.SemaphoreType.DMA,
        pltpu.SemaphoreType.DMA,
        pltpu.SemaphoreType.DMA,
        pltpu.SemaphoreType.DMA,
    ],
    name="similarity_gather_dot_sc",
)
def _sc_gather_dot(x_hbm, sim_hbm, idx_hbm, out_hbm, *scratch):
    _sc_body(x_hbm, sim_hbm, idx_hbm, out_hbm, *scratch)


def _tc_rows_body(t_ref, sim_ref, x_ref, o_ref):
    i = pl.program_id(0)

    @pl.when(i == 0)
    def _init():
        o_ref[0, 0] = 0.0

    x = x_ref[0]
    r = sim_ref[0]
    dot = jnp.sum(x * r)
    sxx = jnp.sum(x * x)
    srr = jnp.sum(r * r)
    x_norm = jnp.sqrt(sxx)
    nx = jnp.maximum(x_norm, 1e-12)
    num = dot / nx
    xn_norm = x_norm / nx
    den = jnp.maximum(xn_norm * jnp.sqrt(srr), 1e-8)
    o_ref[0, 0] += 1.0 - num / den


_tc_rows = pl.pallas_call(
    _tc_rows_body,
    grid_spec=pltpu.PrefetchScalarGridSpec(
        num_scalar_prefetch=1,
        grid=(N_TC,),
        in_specs=[
            pl.BlockSpec((1, 1, V), lambda i, t: (t[i], 0, 0)),
            pl.BlockSpec((1, 1, D), lambda i, t: (i, 0, 0)),
        ],
        out_specs=pl.BlockSpec((1, 1), lambda i, t: (0, 0),
                               memory_space=pltpu.SMEM),
    ),
    out_shape=jax.ShapeDtypeStruct((1, 1), jnp.float32),
    name="similarity_rows_tc",
)


def _epilogue_body(p_ref, tsum_ref, o_ref):
    p = p_ref[...]
    s = jnp.sum(p, axis=-1)
    dot = s[:, 0, :]
    sxx = s[:, 1, :]
    srr = s[:, 2, :]
    x_norm = jnp.sqrt(sxx)
    nx = jnp.maximum(x_norm, 1e-12)
    num = dot / nx
    xn_norm = x_norm / nx
    den = jnp.maximum(xn_norm * jnp.sqrt(srr), 1e-8)
    cos = num / den
    sc_sum = jnp.sum(1.0 - cos)
    o_ref[0, 0] = (sc_sum + tsum_ref[0, 0]) / R


def kernel(logits, sim_matrix, targets):
    x = logits.reshape(R, D)
    t = targets.reshape(-1).astype(jnp.int32)
    t_sc = t[:K_SC].reshape(NW, NCH, CH)
    part = _sc_gather_dot(x[:K_SC], sim_matrix, t_sc)
    tc_sum = _tc_rows(t[K_SC:], sim_matrix.reshape(V, 1, V),
                      x[K_SC:].reshape(N_TC, 1, D))
    loss = pl.pallas_call(
        _epilogue_body,
        out_shape=jax.ShapeDtypeStruct((1, 1), jnp.float32),
        in_specs=[
            pl.BlockSpec(memory_space=pltpu.VMEM),
            pl.BlockSpec(memory_space=pltpu.SMEM),
        ],
        out_specs=pl.BlockSpec(memory_space=pltpu.SMEM),
        name="similarity_loss_epilogue_tc",
    )(part, tc_sum)
    return loss[0, 0]


# speedup vs baseline: 1.9478x; 1.9478x over previous
import functools

import jax
import jax.numpy as jnp
from jax import lax
from jax.experimental import pallas as pl
from jax.experimental.pallas import tpu as pltpu
from jax.experimental.pallas import tpu_sc as plsc

V = 8192
D = 8192
R = 2048
NC = 2
NS = 16
NW = NC * NS
K_SC = 1024
N_TC = R - K_SC
RPW = K_SC // NW
CH = 2
NCH = RPW // CH
L = 16
NBUF = 2
RG = 8


def _sc_body(x_hbm, sim_hbm, idx_hbm, out_hbm,
             idx_v, simbuf, xbuf, res,
             sem_s0, sem_s1, sem_x0, sem_x1):
    sem_s = (sem_s0, sem_s1)
    sem_x = (sem_x0, sem_x1)
    wid = lax.axis_index("s") * NC + lax.axis_index("c")
    base = wid * RPW

    pltpu.sync_copy(idx_hbm.at[wid], idx_v)

    def issue(c, buf):
        pltpu.async_copy(sim_hbm.at[idx_v.at[c]], simbuf.at[buf], sem_s[buf])
        pltpu.async_copy(x_hbm.at[pl.ds(base + c * CH, CH)], xbuf.at[buf],
                         sem_x[buf])

    def wait_chunk(c, buf):
        pltpu.make_async_copy(sim_hbm.at[idx_v.at[c]], simbuf.at[buf],
                              sem_s[buf]).wait()
        pltpu.make_async_copy(x_hbm.at[pl.ds(base + c * CH, CH)],
                              xbuf.at[buf], sem_x[buf]).wait()

    def compute(c, buf):
        for r in range(CH):
            row = c * CH + r

            def body(i, carry):
                sxr, sxx, srr = carry
                off = i * L
                xv = xbuf[buf, r, pl.ds(off, L)]
                rv = simbuf[buf, r, pl.ds(off, L)]
                return (sxr + xv * rv, sxx + xv * xv, srr + rv * rv)

            z = jnp.zeros((L,), jnp.float32)
            sxr, sxx, srr = lax.fori_loop(0, D // L, body, (z, z, z),
                                          unroll=8)
            res[0, row] = sxr
            res[1, row] = sxx
            res[2, row] = srr

    issue(0, 0)
    issue(1, 1)

    def chunk_pair(g, carry):
        c0 = g * NBUF
        for buf in range(NBUF):
            c = c0 + buf
            wait_chunk(c, buf)
            compute(c, buf)

            @pl.when(c + NBUF < NCH)
            def _refill():
                issue(c + NBUF, buf)
        return carry

    lax.fori_loop(0, NCH // NBUF, chunk_pair, 0)
    pltpu.sync_copy(res, out_hbm.at[wid])


@functools.partial(
    pl.kernel,
    out_type=jax.ShapeDtypeStruct((NW, 3, RPW, L), jnp.float32),
    mesh=plsc.VectorSubcoreMesh(core_axis_name="c", subcore_axis_name="s"),
    scratch_types=[
        pltpu.VMEM((NCH, CH), jnp.int32),
        pltpu.VMEM((NBUF, CH, D), jnp.float32),
        pltpu.VMEM((NBUF, CH, D), jnp.float32),
        pltpu.VMEM((3, RPW, L), jnp.float32),
        pltpu.SemaphoreType.DMA,
        pltpu.SemaphoreType.DMA,
        pltpu.SemaphoreType.DMA,
        pltpu.SemaphoreType.DMA,
    ],
    name="similarity_gather_dot_sc",
)
def _sc_gather_dot(x_hbm, sim_hbm, idx_hbm, out_hbm, *scratch):
    _sc_body(x_hbm, sim_hbm, idx_hbm, out_hbm, *scratch)


def _tc_rows_body(t_ref, *refs):
    sim_refs = refs[:RG]
    x_ref, o_ref = refs[RG], refs[RG + 1]
    i = pl.program_id(0)

    @pl.when(i == 0)
    def _init():
        o_ref[0, 0] = 0.0

    acc = 0.0
    for j in range(RG):
        x = x_ref[0, j, :]
        r = sim_refs[j][0, 0, :]
        dot = jnp.sum(x * r)
        sxx = jnp.sum(x * x)
        srr = jnp.sum(r * r)
        x_norm = jnp.sqrt(sxx)
        nx = jnp.maximum(x_norm, 1e-12)
        num = dot / nx
        xn_norm = x_norm / nx
        den = jnp.maximum(xn_norm * jnp.sqrt(srr), 1e-8)
        acc += 1.0 - num / den
    o_ref[0, 0] += acc


def _sim_spec(j):
    return pl.BlockSpec((1, 1, V), lambda i, t, j=j: (t[i * RG + j], 0, 0))


_tc_rows = pl.pallas_call(
    _tc_rows_body,
    grid_spec=pltpu.PrefetchScalarGridSpec(
        num_scalar_prefetch=1,
        grid=(N_TC // RG,),
        in_specs=[_sim_spec(j) for j in range(RG)]
        + [pl.BlockSpec((1, RG, D), lambda i, t: (i, 0, 0))],
        out_specs=pl.BlockSpec((1, 1), lambda i, t: (0, 0),
                               memory_space=pltpu.SMEM),
    ),
    out_shape=jax.ShapeDtypeStruct((1, 1), jnp.float32),
    name="similarity_rows_tc",
)


def _epilogue_body(p_ref, tsum_ref, o_ref):
    p = p_ref[...]
    s = jnp.sum(p, axis=-1)
    dot = s[:, 0, :]
    sxx = s[:, 1, :]
    srr = s[:, 2, :]
    x_norm = jnp.sqrt(sxx)
    nx = jnp.maximum(x_norm, 1e-12)
    num = dot / nx
    xn_norm = x_norm / nx
    den = jnp.maximum(xn_norm * jnp.sqrt(srr), 1e-8)
    cos = num / den
    sc_sum = jnp.sum(1.0 - cos)
    o_ref[0, 0] = (sc_sum + tsum_ref[0, 0]) / R


def kernel(logits, sim_matrix, targets):
    x = logits.reshape(R, D)
    t = targets.reshape(-1).astype(jnp.int32)
    t_sc = t[:K_SC].reshape(NW, NCH, CH)
    part = _sc_gather_dot(x[:K_SC], sim_matrix, t_sc)
    sim3 = sim_matrix.reshape(V, 1, V)
    tc_sum = _tc_rows(t[K_SC:], *([sim3] * RG),
                      x[K_SC:].reshape(N_TC // RG, RG, D))
    loss = pl.pallas_call(
        _epilogue_body,
        out_shape=jax.ShapeDtypeStruct((1, 1), jnp.float32),
        in_specs=[
            pl.BlockSpec(memory_space=pltpu.VMEM),
            pl.BlockSpec(memory_space=pltpu.SMEM),
        ],
        out_specs=pl.BlockSpec(memory_space=pltpu.SMEM),
        name="similarity_loss_epilogue_tc",
    )(part, tc_sum)
    return loss[0, 0]


# speedup vs baseline: 2.2597x; 1.1601x over previous
import functools

import jax
import jax.numpy as jnp
from jax import lax
from jax.experimental import pallas as pl
from jax.experimental.pallas import tpu as pltpu
from jax.experimental.pallas import tpu_sc as plsc

V = 8192
D = 8192
R = 2048
NC = 2
NS = 16
NW = NC * NS
K_SC = 1024
N_TC = R - K_SC
RPW = K_SC // NW
CH = 2
NCH = RPW // CH
L = 16
NBUF = 2
RG = 8


def _sc_body(x_hbm, sim_hbm, idx_hbm, out_hbm,
             idx_v, simbuf, xbuf, res,
             sem_s0, sem_s1, sem_x0, sem_x1):
    sem_s = (sem_s0, sem_s1)
    sem_x = (sem_x0, sem_x1)
    wid = lax.axis_index("s") * NC + lax.axis_index("c")
    base = wid * RPW

    pltpu.sync_copy(idx_hbm.at[wid], idx_v)

    def issue(c, buf):
        pltpu.async_copy(sim_hbm.at[idx_v.at[c]], simbuf.at[buf], sem_s[buf])
        pltpu.async_copy(x_hbm.at[pl.ds(base + c * CH, CH)], xbuf.at[buf],
                         sem_x[buf])

    def wait_chunk(c, buf):
        pltpu.make_async_copy(sim_hbm.at[idx_v.at[c]], simbuf.at[buf],
                              sem_s[buf]).wait()
        pltpu.make_async_copy(x_hbm.at[pl.ds(base + c * CH, CH)],
                              xbuf.at[buf], sem_x[buf]).wait()

    def compute(c, buf):
        for r in range(CH):
            row = c * CH + r

            def body(i, carry):
                sxr, sxx, srr = carry
                off = i * L
                xv = xbuf[buf, r, pl.ds(off, L)]
                rv = simbuf[buf, r, pl.ds(off, L)]
                return (sxr + xv * rv, sxx + xv * xv, srr + rv * rv)

            z = jnp.zeros((L,), jnp.float32)
            sxr, sxx, srr = lax.fori_loop(0, D // L, body, (z, z, z),
                                          unroll=8)
            res[0, row] = sxr
            res[1, row] = sxx
            res[2, row] = srr

    issue(0, 0)
    issue(1, 1)

    def chunk_pair(g, carry):
        c0 = g * NBUF
        for buf in range(NBUF):
            c = c0 + buf
            wait_chunk(c, buf)
            compute(c, buf)

            @pl.when(c + NBUF < NCH)
            def _refill():
                issue(c + NBUF, buf)
        return carry

    lax.fori_loop(0, NCH // NBUF, chunk_pair, 0)
    pltpu.sync_copy(res, out_hbm.at[wid])


@functools.partial(
    pl.kernel,
    out_type=jax.ShapeDtypeStruct((NW, 3, RPW, L), jnp.float32),
    mesh=plsc.VectorSubcoreMesh(core_axis_name="c", subcore_axis_name="s"),
    scratch_types=[
        pltpu.VMEM((NCH, CH), jnp.int32),
        pltpu.VMEM((NBUF, CH, D), jnp.float32),
        pltpu.VMEM((NBUF, CH, D), jnp.float32),
        pltpu.VMEM((3, RPW, L), jnp.float32),
        pltpu.SemaphoreType.DMA,
        pltpu.SemaphoreType.DMA,
        pltpu.SemaphoreType.DMA,
        pltpu.SemaphoreType.DMA,
    ],
    name="similarity_gather_dot_sc",
)
def _sc_gather_dot(x_hbm, sim_hbm, idx_hbm, out_hbm, *scratch):
    _sc_body(x_hbm, sim_hbm, idx_hbm, out_hbm, *scratch)


def _tc_rows_body(t_ref, *refs):
    sim_refs = refs[:RG]
    x_ref, o_ref = refs[RG], refs[RG + 1]
    i = pl.program_id(0)

    @pl.when(i == 0)
    def _init():
        o_ref[0, 0] = 0.0

    x = x_ref[0]
    r = jnp.concatenate([s[0] for s in sim_refs], axis=0)
    dot = jnp.sum(x * r, axis=1)
    sxx = jnp.sum(x * x, axis=1)
    srr = jnp.sum(r * r, axis=1)
    x_norm = jnp.sqrt(sxx)
    nx = jnp.maximum(x_norm, 1e-12)
    num = dot / nx
    xn_norm = x_norm / nx
    den = jnp.maximum(xn_norm * jnp.sqrt(srr), 1e-8)
    o_ref[0, 0] += jnp.sum(1.0 - num / den)


def _sim_spec(j):
    return pl.BlockSpec((1, 1, V), lambda i, t, j=j: (t[i * RG + j], 0, 0))


_tc_rows = pl.pallas_call(
    _tc_rows_body,
    grid_spec=pltpu.PrefetchScalarGridSpec(
        num_scalar_prefetch=1,
        grid=(N_TC // RG,),
        in_specs=[_sim_spec(j) for j in range(RG)]
        + [pl.BlockSpec((1, RG, D), lambda i, t: (i, 0, 0))],
        out_specs=pl.BlockSpec((1, 1), lambda i, t: (0, 0),
                               memory_space=pltpu.SMEM),
    ),
    out_shape=jax.ShapeDtypeStruct((1, 1), jnp.float32),
    name="similarity_rows_tc",
)


def _epilogue_body(p_ref, tsum_ref, o_ref):
    p = p_ref[...]
    s = jnp.sum(p, axis=-1)
    dot = s[:, 0, :]
    sxx = s[:, 1, :]
    srr = s[:, 2, :]
    x_norm = jnp.sqrt(sxx)
    nx = jnp.maximum(x_norm, 1e-12)
    num = dot / nx
    xn_norm = x_norm / nx
    den = jnp.maximum(xn_norm * jnp.sqrt(srr), 1e-8)
    cos = num / den
    sc_sum = jnp.sum(1.0 - cos)
    o_ref[0, 0] = (sc_sum + tsum_ref[0, 0]) / R


def kernel(logits, sim_matrix, targets):
    x = logits.reshape(R, D)
    t = targets.reshape(-1).astype(jnp.int32)
    t_sc = t[:K_SC].reshape(NW, NCH, CH)
    part = _sc_gather_dot(x[:K_SC], sim_matrix, t_sc)
    sim3 = sim_matrix.reshape(V, 1, V)
    tc_sum = _tc_rows(t[K_SC:], *([sim3] * RG),
                      x[K_SC:].reshape(N_TC // RG, RG, D))
    loss = pl.pallas_call(
        _epilogue_body,
        out_shape=jax.ShapeDtypeStruct((1, 1), jnp.float32),
        in_specs=[
            pl.BlockSpec(memory_space=pltpu.VMEM),
            pl.BlockSpec(memory_space=pltpu.SMEM),
        ],
        out_specs=pl.BlockSpec(memory_space=pltpu.SMEM),
        name="similarity_loss_epilogue_tc",
    )(part, tc_sum)
    return loss[0, 0]


# speedup vs baseline: 14.6086x; 6.4648x over previous
import functools

import jax
import jax.numpy as jnp
from jax import lax
from jax.experimental import pallas as pl
from jax.experimental.pallas import tpu as pltpu
from jax.experimental.pallas import tpu_sc as plsc

V = 8192
D = 8192
R = 2048
NC = 2
NS = 16
NW = NC * NS
RPW = R // NW
L = 16
NBUF = 4


def _sc_body(x_hbm, sim_hbm, idx_hbm, out_hbm,
             idx_v, simbuf, xbuf, res, sem_s, sem_x):
    wid = lax.axis_index("s") * NC + lax.axis_index("c")
    base = wid * RPW

    pltpu.sync_copy(idx_hbm.at[wid], idx_v)

    def issue(c, buf):
        pltpu.async_copy(sim_hbm.at[idx_v.at[c]], simbuf.at[buf],
                         sem_s.at[buf])
        pltpu.async_copy(x_hbm.at[pl.ds(base + c, 1)], xbuf.at[buf],
                         sem_x.at[buf])

    def wait_row(c, buf):
        pltpu.make_async_copy(sim_hbm.at[idx_v.at[c]], simbuf.at[buf],
                              sem_s.at[buf]).wait()
        pltpu.make_async_copy(x_hbm.at[pl.ds(base + c, 1)], xbuf.at[buf],
                              sem_x.at[buf]).wait()

    def compute(c, buf):
        def body(i, carry):
            sxr, sxx, srr = carry
            off = i * L
            xv = xbuf[buf, 0, pl.ds(off, L)]
            rv = simbuf[buf, 0, pl.ds(off, L)]
            return (sxr + xv * rv, sxx + xv * xv, srr + rv * rv)

        z = jnp.zeros((L,), jnp.float32)
        sxr, sxx, srr = lax.fori_loop(0, D // L, body, (z, z, z), unroll=16)
        res[0, c] = sxr
        res[1, c] = sxx
        res[2, c] = srr

    for buf in range(NBUF):
        issue(buf, buf)

    def ring_step(g, carry):
        c0 = g * NBUF
        for buf in range(NBUF):
            c = c0 + buf
            wait_row(c, buf)
            compute(c, buf)

            @pl.when(c + NBUF < RPW)
            def _refill():
                issue(c + NBUF, buf)
        return carry

    lax.fori_loop(0, RPW // NBUF, ring_step, 0)
    pltpu.sync_copy(res, out_hbm.at[wid])


@functools.partial(
    pl.kernel,
    out_type=jax.ShapeDtypeStruct((NW, 3, RPW, L), jnp.float32),
    mesh=plsc.VectorSubcoreMesh(core_axis_name="c", subcore_axis_name="s"),
    scratch_types=[
        pltpu.VMEM((RPW, 1), jnp.int32),
        pltpu.VMEM((NBUF, 1, D), jnp.float32),
        pltpu.VMEM((NBUF, 1, D), jnp.float32),
        pltpu.VMEM((3, RPW, L), jnp.float32),
        pltpu.SemaphoreType.DMA((NBUF,)),
        pltpu.SemaphoreType.DMA((NBUF,)),
    ],
    name="similarity_gather_dot_sc",
)
def _sc_gather_dot(x_hbm, sim_hbm, idx_hbm, out_hbm, *scratch):
    _sc_body(x_hbm, sim_hbm, idx_hbm, out_hbm, *scratch)


def _epilogue_body(p_ref, o_ref):
    p = p_ref[...]
    s = jnp.sum(p, axis=-1)
    dot = s[:, 0, :]
    sxx = s[:, 1, :]
    srr = s[:, 2, :]
    x_norm = jnp.sqrt(sxx)
    nx = jnp.maximum(x_norm, 1e-12)
    num = dot / nx
    xn_norm = x_norm / nx
    den = jnp.maximum(xn_norm * jnp.sqrt(srr), 1e-8)
    cos = num / den
    o_ref[0, 0] = jnp.mean(1.0 - cos)


def kernel(logits, sim_matrix, targets):
    x = logits.reshape(R, D)
    t = targets.reshape(-1).astype(jnp.int32).reshape(NW, RPW, 1)
    part = _sc_gather_dot(x, sim_matrix, t)
    loss = pl.pallas_call(
        _epilogue_body,
        out_shape=jax.ShapeDtypeStruct((1, 1), jnp.float32),
        in_specs=[pl.BlockSpec(memory_space=pltpu.VMEM)],
        out_specs=pl.BlockSpec(memory_space=pltpu.SMEM),
        name="similarity_loss_epilogue_tc",
    )(part)
    return loss[0, 0]


# speedup vs baseline: 14.8362x; 1.0156x over previous
import functools

import jax
import jax.numpy as jnp
from jax import lax
from jax.experimental import pallas as pl
from jax.experimental.pallas import tpu as pltpu
from jax.experimental.pallas import tpu_sc as plsc

V = 8192
D = 8192
B = 64
S = 32
R = 2048
NC = 2
NS = 16
NW = NC * NS
RPW = R // NW
L = 16
NBUF = 4


def _sc_body(x_hbm, sim_hbm, idx_hbm, out_hbm,
             idx_v, simbuf, xbuf, res, sem_s, sem_x):
    wid = lax.axis_index("s") * NC + lax.axis_index("c")
    base = wid * RPW

    pltpu.sync_copy(idx_hbm.at[pl.ds(wid * (RPW // S), RPW // S)], idx_v)

    def _idx(c):
        return idx_v.at[c // S, pl.ds(lax.rem(c, S), 1)]

    def issue(c, buf):
        pltpu.async_copy(sim_hbm.at[_idx(c)], simbuf.at[buf],
                         sem_s.at[buf])
        pltpu.async_copy(x_hbm.at[pl.ds(base + c, 1)], xbuf.at[buf],
                         sem_x.at[buf])

    def wait_row(c, buf):
        pltpu.make_async_copy(sim_hbm.at[_idx(c)], simbuf.at[buf],
                              sem_s.at[buf]).wait()
        pltpu.make_async_copy(x_hbm.at[pl.ds(base + c, 1)], xbuf.at[buf],
                              sem_x.at[buf]).wait()

    def compute(c, buf):
        def body(i, carry):
            sxr, sxx, srr = carry
            off = i * L
            xv = xbuf[buf, 0, pl.ds(off, L)]
            rv = simbuf[buf, 0, pl.ds(off, L)]
            return (sxr + xv * rv, sxx + xv * xv, srr + rv * rv)

        z = jnp.zeros((L,), jnp.float32)
        sxr, sxx, srr = lax.fori_loop(0, D // L, body, (z, z, z), unroll=8)
        res[0, c] = sxr
        res[1, c] = sxx
        res[2, c] = srr

    for buf in range(NBUF):
        issue(buf, buf)

    def ring_step(g, carry):
        c0 = g * NBUF
        for buf in range(NBUF):
            c = c0 + buf
            wait_row(c, buf)
            compute(c, buf)

            @pl.when(c + NBUF < RPW)
            def _refill():
                issue(c + NBUF, buf)
        return carry

    lax.fori_loop(0, RPW // NBUF, ring_step, 0)
    pltpu.sync_copy(res, out_hbm.at[wid])


@functools.partial(
    pl.kernel,
    out_type=jax.ShapeDtypeStruct((NW, 3, RPW, L), jnp.float32),
    mesh=plsc.VectorSubcoreMesh(core_axis_name="c", subcore_axis_name="s"),
    scratch_types=[
        pltpu.VMEM((RPW // S, S), jnp.int32),
        pltpu.VMEM((NBUF, 1, D), jnp.float32),
        pltpu.VMEM((NBUF, 1, D), jnp.float32),
        pltpu.VMEM((3, RPW, L), jnp.float32),
        pltpu.SemaphoreType.DMA((NBUF,)),
        pltpu.SemaphoreType.DMA((NBUF,)),
    ],
    name="similarity_gather_dot_sc",
)
def _sc_gather_dot(x_hbm, sim_hbm, idx_hbm, out_hbm, *scratch):
    _sc_body(x_hbm, sim_hbm, idx_hbm, out_hbm, *scratch)


def _epilogue_body(p_ref, o_ref):
    p = p_ref[...]
    s = jnp.sum(p, axis=-1)
    dot = s[:, 0, :]
    sxx = s[:, 1, :]
    srr = s[:, 2, :]
    x_norm = jnp.sqrt(sxx)
    nx = jnp.maximum(x_norm, 1e-12)
    num = dot / nx
    xn_norm = x_norm / nx
    den = jnp.maximum(xn_norm * jnp.sqrt(srr), 1e-8)
    cos = num / den
    o_ref[0, 0] = jnp.mean(1.0 - cos)


def kernel(logits, sim_matrix, targets):
    x = logits.reshape(R, D)
    part = _sc_gather_dot(x, sim_matrix, targets.astype(jnp.int32))
    loss = pl.pallas_call(
        _epilogue_body,
        out_shape=jax.ShapeDtypeStruct((1, 1), jnp.float32),
        in_specs=[pl.BlockSpec(memory_space=pltpu.VMEM)],
        out_specs=pl.BlockSpec(memory_space=pltpu.SMEM),
        name="similarity_loss_epilogue_tc",
    )(part)
    return loss[0, 0]
